# baseline (speedup 1.0000x reference)
import functools

import jax
import jax.numpy as jnp
from jax import lax
from jax.experimental import pallas as pl
from jax.experimental.pallas import tpu as pltpu
from jax.experimental.pallas import tpu_sc as plsc

B, S = 16, 1500
CODES, SW, W = 1024, 768, 1024
NT = B * S
SCK = 2
C = SCK * B
NCHUNKS = NT // C
NWORKERS = 32
LANES = 16
KMAX = (NCHUNKS + NWORKERS - 1) // NWORKERS
NFULL = NCHUNKS - (KMAX - 1) * NWORKERS
SPAD = 1504


def _mm_body(a_ref, b_ref, bias_ref, o_ref):
    o_ref[...] = (
        jnp.dot(a_ref[...], b_ref[...], preferred_element_type=jnp.float32,
                precision=lax.Precision.HIGHEST)
        + bias_ref[...]
    )


def _project_table(main_w, e2h_w, e2h_b):
    return pl.pallas_call(
        _mm_body,
        out_shape=jax.ShapeDtypeStruct((CODES, W), jnp.float32),
    )(main_w, e2h_w, e2h_b.reshape(1, W))


def _sc_body(table, idxs, oidx, pos_emb, out,
             idx_v, oidx_v, pos_v, rows_v, sem_g, sem_s):
    wid = lax.axis_index("s") * 2 + lax.axis_index("c")
    is_full = wid < NFULL

    def issue(k, p):
        c = wid + NWORKERS * k
        base = c * C
        s0 = c * SCK
        pltpu.sync_copy(idxs.at[pl.ds(base, C)], idx_v[p])
        pltpu.sync_copy(oidx.at[pl.ds(base, C)], oidx_v[p])
        pltpu.sync_copy(pos_emb.at[pl.ds((s0 // 8) * 8, 8)], pos_v[p])
        pltpu.async_copy(table.at[idx_v[p]], rows_v[p], sem_g[p])

    def wait_gather(p):
        pltpu.make_async_copy(table.at[idx_v[p]], rows_v[p], sem_g[p]).wait()

    def wait_scatter(p):
        pltpu.make_async_copy(rows_v[p], out.at[oidx_v[p]], sem_s[p]).wait()

    def add_pos(k, p):
        c = wid + NWORKERS * k
        d = c * SCK - (c * SCK // 8) * 8

        def row_add(i, carry):
            sl = d + i // B
            for j in range(W // LANES):
                plsc.addupdate(rows_v[p].at[i, pl.ds(LANES * j, LANES)],
                               pos_v[p][sl, pl.ds(LANES * j, LANES)])
            return carry
        lax.fori_loop(0, C, row_add, 0, unroll=False)

    issue(0, 0)
    for k in range(KMAX):
        p = k % 2
        q = 1 - p
        guard_k = is_full if k == KMAX - 1 else None
        if k + 1 < KMAX:
            if k >= 1:
                wait_scatter(q)
            if k + 1 == KMAX - 1:
                @pl.when(is_full)
                def _():
                    issue(k + 1, q)
            else:
                issue(k + 1, q)
        if guard_k is None:
            wait_gather(p)
            add_pos(k, p)
            pltpu.async_copy(rows_v[p], out.at[oidx_v[p]], sem_s[p])
        else:
            @pl.when(guard_k)
            def _():
                wait_gather(p)
                add_pos(k, p)
                pltpu.async_copy(rows_v[p], out.at[oidx_v[p]], sem_s[p])
    wait_scatter((KMAX - 2) % 2)

    @pl.when(is_full)
    def _():
        wait_scatter((KMAX - 1) % 2)


@functools.partial(
    pl.kernel,
    out_type=jax.ShapeDtypeStruct((NT, W), jnp.float32),
    mesh=plsc.VectorSubcoreMesh(core_axis_name="c", subcore_axis_name="s"),
    scratch_types=[
        [pltpu.VMEM((C,), jnp.int32)] * 2,
        [pltpu.VMEM((C,), jnp.int32)] * 2,
        [pltpu.VMEM((8, W), jnp.float32)] * 2,
        [pltpu.VMEM((C, W), jnp.float32)] * 2,
        [pltpu.SemaphoreType.DMA] * 2,
        [pltpu.SemaphoreType.DMA] * 2,
    ],
)
def _sc_gather_add(table, idxs, oidx, pos_emb, out,
                   idx_v, oidx_v, pos_v, rows_v, sem_g, sem_s):
    _sc_body(table, idxs, oidx, pos_emb, out,
             idx_v, oidx_v, pos_v, rows_v, sem_g, sem_s)


def kernel(Stoks, xenc, main_w, special_w, e2h_w, e2h_b, pos_emb):
    proj = _project_table(main_w, e2h_w, e2h_b)
    table = jnp.concatenate([proj, special_w], axis=0)
    pos_pad = jnp.pad(pos_emb, ((0, SPAD - S), (0, 0)))
    idxs = jnp.transpose(Stoks).reshape(NT).astype(jnp.int32)
    t = jnp.arange(NT, dtype=jnp.int32)
    oidx = (t % B) * S + t // B
    out = _sc_gather_add(table, idxs, oidx, pos_pad)
    return (out.reshape(B, S, W).astype(xenc.dtype), 0)


# speedup vs baseline: 1.7538x; 1.7538x over previous
import functools

import jax
import jax.numpy as jnp
from jax import lax
from jax.experimental import pallas as pl
from jax.experimental.pallas import tpu as pltpu
from jax.experimental.pallas import tpu_sc as plsc

B, S = 16, 1500
CODES, SW, W = 1024, 768, 1024
NT = B * S
NWORKERS = 32
LANES = 16
TPT = 768
SPT = TPT // B
C = 32
KFULL = TPT // C
KLAST = (NT - 31 * TPT) // C


def _mm_body(a_ref, b_ref, bias_ref, o_ref):
    o_ref[...] = (
        jnp.dot(a_ref[...], b_ref[...], preferred_element_type=jnp.float32,
                precision=lax.Precision.HIGHEST)
        + bias_ref[...]
    )


def _project_table(main_w, e2h_w, e2h_b):
    return pl.pallas_call(
        _mm_body,
        out_shape=jax.ShapeDtypeStruct((CODES, W), jnp.float32),
    )(main_w, e2h_w, e2h_b.reshape(1, W))


def _sc_body(table, idxs, pos_emb, out, idx_v, pos_v, rows_v, sem_g, sem_w):
    wid = lax.axis_index("s") * 2 + lax.axis_index("c")
    t0 = wid * TPT
    is_last = wid == NWORKERS - 1

    pltpu.sync_copy(idxs.at[pl.ds(t0, TPT)], idx_v)
    pltpu.sync_copy(pos_emb.at[pl.ds(wid * SPT, SPT)], pos_v)

    def gather(k, p):
        pltpu.async_copy(table.at[idx_v.at[pl.ds(C * k, C)]], rows_v[p], sem_g[p])

    def wait_gather(k, p):
        pltpu.make_async_copy(table.at[idx_v.at[pl.ds(C * k, C)]],
                              rows_v[p], sem_g[p]).wait()

    def write(k, p):
        pltpu.async_copy(rows_v[p], out.at[pl.ds(t0 + C * k, C)], sem_w[p])

    def wait_write(k, p):
        pltpu.make_async_copy(rows_v[p], out.at[pl.ds(t0 + C * k, C)],
                              sem_w[p]).wait()

    def add_pos(k, p):
        def row_add(i, carry):
            sl = 2 * k + i // B
            for j in range(W // LANES):
                plsc.addupdate(rows_v[p].at[i, pl.ds(LANES * j, LANES)],
                               pos_v[sl, pl.ds(LANES * j, LANES)])
            return carry
        lax.fori_loop(0, C, row_add, 0, unroll=False)

    def chunk_tail(k, p):
        wait_gather(k, p)
        add_pos(k, p)
        write(k, p)

    gather(0, 0)
    for k in range(KFULL):
        p = k % 2
        q = 1 - p
        if k + 1 < KFULL:
            def prefetch(k=k, q=q):
                if k >= 1:
                    wait_write(k - 1, q)
                gather(k + 1, q)
            if k + 1 < KLAST:
                prefetch()
            else:
                pl.when(~is_last)(prefetch)
        if k < KLAST:
            chunk_tail(k, p)
        else:
            @pl.when(~is_last)
            def _():
                chunk_tail(k, p)
    @pl.when(~is_last)
    def _():
        wait_write(KFULL - 2, (KFULL - 2) % 2)
        wait_write(KFULL - 1, (KFULL - 1) % 2)

    @pl.when(is_last)
    def _():
        wait_write(KLAST - 2, (KLAST - 2) % 2)
        wait_write(KLAST - 1, (KLAST - 1) % 2)


@functools.partial(
    pl.kernel,
    out_type=jax.ShapeDtypeStruct((NT, W), jnp.float32),
    mesh=plsc.VectorSubcoreMesh(core_axis_name="c", subcore_axis_name="s"),
    scratch_types=[
        pltpu.VMEM((TPT,), jnp.int32),
        pltpu.VMEM((SPT, W), jnp.float32),
        [pltpu.VMEM((C, W), jnp.float32)] * 2,
        [pltpu.SemaphoreType.DMA] * 2,
        [pltpu.SemaphoreType.DMA] * 2,
    ],
)
def _sc_gather_add(table, idxs, pos_emb, out, idx_v, pos_v, rows_v, sem_g, sem_w):
    _sc_body(table, idxs, pos_emb, out, idx_v, pos_v, rows_v, sem_g, sem_w)


def kernel(Stoks, xenc, main_w, special_w, e2h_w, e2h_b, pos_emb):
    proj = _project_table(main_w, e2h_w, e2h_b)
    table = jnp.concatenate([proj, special_w], axis=0)
    pos_pad = jnp.pad(pos_emb, ((0, NWORKERS * SPT - S), (0, 0)))
    idxs = jnp.transpose(Stoks).reshape(NT).astype(jnp.int32)
    out = _sc_gather_add(table, idxs, pos_pad)
    xin = jnp.transpose(out.reshape(S, B, W), (1, 0, 2))
    return (xin.astype(xenc.dtype), 0)


# speedup vs baseline: 1.9493x; 1.1115x over previous
import functools

import jax
import jax.numpy as jnp
from jax import lax
from jax.experimental import pallas as pl
from jax.experimental.pallas import tpu as pltpu
from jax.experimental.pallas import tpu_sc as plsc

B, S = 16, 1500
CODES, SW, W = 1024, 768, 1024
NT = B * S
NWORKERS = 32
LANES = 16
TPT = 768
SPT = TPT // B
C = 24
NBUF = 3
KFULL = TPT // C
KLAST = (NT - 31 * TPT) // C


def _mm_body(a_ref, b_ref, bias_ref, o_ref):
    o_ref[...] = (
        jnp.dot(a_ref[...], b_ref[...], preferred_element_type=jnp.float32,
                precision=lax.Precision.HIGHEST)
        + bias_ref[...]
    )


def _project_table(main_w, e2h_w, e2h_b):
    return pl.pallas_call(
        _mm_body,
        out_shape=jax.ShapeDtypeStruct((CODES, W), jnp.float32),
    )(main_w, e2h_w, e2h_b.reshape(1, W))


def _sc_body(table, idxs, pos_emb, out, idx_v, pos_v, rows_v, sem_g, sem_w):
    wid = lax.axis_index("s") * 2 + lax.axis_index("c")
    t0 = wid * TPT
    is_last = wid == NWORKERS - 1

    pltpu.sync_copy(idxs.at[pl.ds(t0, TPT)], idx_v)
    pltpu.sync_copy(pos_emb.at[pl.ds(wid * SPT, SPT)], pos_v)

    def gather(k, p):
        pltpu.async_copy(table.at[idx_v.at[pl.ds(C * k, C)]], rows_v[p], sem_g[p])

    def wait_gather(k, p):
        pltpu.make_async_copy(table.at[idx_v.at[pl.ds(C * k, C)]],
                              rows_v[p], sem_g[p]).wait()

    def write(k, p):
        pltpu.async_copy(rows_v[p], out.at[pl.ds(t0 + C * k, C)], sem_w[p])

    def wait_write(k, p):
        pltpu.make_async_copy(rows_v[p], out.at[pl.ds(t0 + C * k, C)],
                              sem_w[p]).wait()

    def add_pos(k, p):
        def row_add(i, carry):
            sl = (C * k + i) // B
            for j in range(W // LANES):
                plsc.addupdate(rows_v[p].at[i, pl.ds(LANES * j, LANES)],
                               pos_v[sl, pl.ds(LANES * j, LANES)])
            return carry
        lax.fori_loop(0, C, row_add, 0, unroll=False)

    def chunk_tail(k, p):
        wait_gather(k, p)
        add_pos(k, p)
        write(k, p)

    gather(0, 0)
    for k in range(KFULL):
        p = k % NBUF
        if k + 1 < KFULL:
            def prefetch(k=k):
                if k + 1 - NBUF >= 0:
                    wait_write(k + 1 - NBUF, (k + 1) % NBUF)
                gather(k + 1, (k + 1) % NBUF)
            if k + 1 < KLAST:
                prefetch()
            else:
                pl.when(~is_last)(prefetch)
        if k < KLAST:
            chunk_tail(k, p)
        else:
            @pl.when(~is_last)
            def _():
                chunk_tail(k, p)
    @pl.when(~is_last)
    def _():
        for k in range(KFULL - NBUF, KFULL):
            wait_write(k, k % NBUF)

    @pl.when(is_last)
    def _():
        for k in range(max(KLAST - NBUF, 0), KLAST):
            wait_write(k, k % NBUF)


@functools.partial(
    pl.kernel,
    out_type=jax.ShapeDtypeStruct((NT, W), jnp.float32),
    mesh=plsc.VectorSubcoreMesh(core_axis_name="c", subcore_axis_name="s"),
    scratch_types=[
        pltpu.VMEM((TPT,), jnp.int32),
        pltpu.VMEM((SPT, W), jnp.float32),
        [pltpu.VMEM((C, W), jnp.float32)] * NBUF,
        [pltpu.SemaphoreType.DMA] * NBUF,
        [pltpu.SemaphoreType.DMA] * NBUF,
    ],
)
def _sc_gather_add(table, idxs, pos_emb, out, idx_v, pos_v, rows_v, sem_g, sem_w):
    _sc_body(table, idxs, pos_emb, out, idx_v, pos_v, rows_v, sem_g, sem_w)


def kernel(Stoks, xenc, main_w, special_w, e2h_w, e2h_b, pos_emb):
    proj = _project_table(main_w, e2h_w, e2h_b)
    table = jnp.concatenate([proj, special_w], axis=0)
    pos_pad = jnp.pad(pos_emb, ((0, NWORKERS * SPT - S), (0, 0)))
    idxs = jnp.transpose(Stoks).reshape(NT).astype(jnp.int32)
    out = _sc_gather_add(table, idxs, pos_pad)
    xin = jnp.transpose(out.reshape(S, B, W), (1, 0, 2))
    return (xin.astype(xenc.dtype), 0)


# speedup vs baseline: 3.5441x; 1.8181x over previous
import functools

import jax
import jax.numpy as jnp
from jax import lax
from jax.experimental import pallas as pl
from jax.experimental.pallas import tpu as pltpu
from jax.experimental.pallas import tpu_sc as plsc

B, S = 16, 1500
CODES, SW, W = 1024, 768, 1024
NT = B * S
NWORKERS = 32
LANES = 16
TPT = 768
SPT = TPT // B
C = 24
NBUF = 3
KFULL = TPT // C
KLAST = (NT - 31 * TPT) // C


def _mm_body(a_ref, b_ref, bias_ref, o_ref):
    o_ref[...] = (
        jnp.dot(a_ref[...], b_ref[...], preferred_element_type=jnp.float32,
                precision=lax.Precision.HIGHEST)
        + bias_ref[...]
    )


def _project_table(main_w, e2h_w, e2h_b):
    return pl.pallas_call(
        _mm_body,
        out_shape=jax.ShapeDtypeStruct((CODES, W), jnp.float32),
    )(main_w, e2h_w, e2h_b.reshape(1, W))


def _sc_body(table, idxs, pos_emb, out, idx_v, pos_v, rows_v, sem_g, sem_w):
    wid = lax.axis_index("s") * 2 + lax.axis_index("c")
    t0 = wid * TPT
    is_last = wid == NWORKERS - 1

    pltpu.sync_copy(idxs.at[pl.ds(t0, TPT)], idx_v)
    pltpu.sync_copy(pos_emb.at[pl.ds(wid * SPT, SPT)], pos_v)

    def gather(k, p):
        pltpu.async_copy(table.at[idx_v.at[pl.ds(C * k, C)]], rows_v[p], sem_g[p])

    def wait_gather(k, p):
        pltpu.make_async_copy(table.at[idx_v.at[pl.ds(C * k, C)]],
                              rows_v[p], sem_g[p]).wait()

    def write(k, p):
        pltpu.async_copy(rows_v[p], out.at[pl.ds(t0 + C * k, C)], sem_w[p])

    def wait_write(k, p):
        pltpu.make_async_copy(rows_v[p], out.at[pl.ds(t0 + C * k, C)],
                              sem_w[p]).wait()

    def add_pos(k, p):
        m = (C * k) % B
        s0 = (C * k) // B
        if m == 0:
            segs = ((0, B, s0), (B, C - B, s0 + 1))
        else:
            segs = ((0, B - m, s0), (B - m, C - (B - m), s0 + 1))

        def j_body(j, carry):
            off = LANES * j
            for start, ln, sl in segs:
                v = pos_v[sl, pl.ds(off, LANES)]
                for i in range(start, start + ln):
                    plsc.addupdate(rows_v[p].at[i, pl.ds(off, LANES)], v)
            return carry
        lax.fori_loop(0, W // LANES, j_body, 0, unroll=False)

    def chunk_tail(k, p):
        wait_gather(k, p)
        add_pos(k, p)
        write(k, p)

    gather(0, 0)
    for k in range(KFULL):
        p = k % NBUF
        if k + 1 < KFULL:
            def prefetch(k=k):
                if k + 1 - NBUF >= 0:
                    wait_write(k + 1 - NBUF, (k + 1) % NBUF)
                gather(k + 1, (k + 1) % NBUF)
            if k + 1 < KLAST:
                prefetch()
            else:
                pl.when(~is_last)(prefetch)
        if k < KLAST:
            chunk_tail(k, p)
        else:
            @pl.when(~is_last)
            def _():
                chunk_tail(k, p)
    @pl.when(~is_last)
    def _():
        for k in range(KFULL - NBUF, KFULL):
            wait_write(k, k % NBUF)

    @pl.when(is_last)
    def _():
        for k in range(max(KLAST - NBUF, 0), KLAST):
            wait_write(k, k % NBUF)


@functools.partial(
    pl.kernel,
    out_type=jax.ShapeDtypeStruct((NT, W), jnp.float32),
    mesh=plsc.VectorSubcoreMesh(core_axis_name="c", subcore_axis_name="s"),
    scratch_types=[
        pltpu.VMEM((TPT,), jnp.int32),
        pltpu.VMEM((SPT, W), jnp.float32),
        [pltpu.VMEM((C, W), jnp.float32)] * NBUF,
        [pltpu.SemaphoreType.DMA] * NBUF,
        [pltpu.SemaphoreType.DMA] * NBUF,
    ],
)
def _sc_gather_add(table, idxs, pos_emb, out, idx_v, pos_v, rows_v, sem_g, sem_w):
    _sc_body(table, idxs, pos_emb, out, idx_v, pos_v, rows_v, sem_g, sem_w)


def kernel(Stoks, xenc, main_w, special_w, e2h_w, e2h_b, pos_emb):
    proj = _project_table(main_w, e2h_w, e2h_b)
    table = jnp.concatenate([proj, special_w], axis=0)
    pos_pad = jnp.pad(pos_emb, ((0, NWORKERS * SPT - S), (0, 0)))
    idxs = jnp.transpose(Stoks).reshape(NT).astype(jnp.int32)
    out = _sc_gather_add(table, idxs, pos_pad)
    xin = jnp.transpose(out.reshape(S, B, W), (1, 0, 2))
    return (xin.astype(xenc.dtype), 0)


# speedup vs baseline: 3.5493x; 1.0015x over previous
import functools

import jax
import jax.numpy as jnp
from jax import lax
from jax.experimental import pallas as pl
from jax.experimental.pallas import tpu as pltpu
from jax.experimental.pallas import tpu_sc as plsc

B, S = 16, 1500
CODES, SW, W = 1024, 768, 1024
NT = B * S
NWORKERS = 32
LANES = 16
TPT = 768
SPT = TPT // B
C = 24
NBUF = 3
KFULL = TPT // C
KLAST = (NT - 31 * TPT) // C


def _mm_body(a_ref, b_ref, bias_ref, o_ref):
    o_ref[...] = (
        jnp.dot(a_ref[...], b_ref[...], preferred_element_type=jnp.float32,
                precision=lax.Precision.HIGHEST)
        + bias_ref[...]
    )


def _project_table(main_w, e2h_w, e2h_b):
    return pl.pallas_call(
        _mm_body,
        out_shape=jax.ShapeDtypeStruct((CODES, W), jnp.float32),
    )(main_w, e2h_w, e2h_b.reshape(1, W))


def _sc_body(table, idxs, pos_emb, out, idx_v, pos_v, rows_v, sem_g, sem_w):
    wid = lax.axis_index("s") * 2 + lax.axis_index("c")
    t0 = wid * TPT
    is_last = wid == NWORKERS - 1

    pltpu.sync_copy(idxs.at[pl.ds(t0, TPT)], idx_v)
    pltpu.sync_copy(pos_emb.at[pl.ds(wid * SPT, SPT)], pos_v)

    def gather(k, p):
        pltpu.async_copy(table.at[idx_v.at[pl.ds(C * k, C)]], rows_v[p], sem_g[p])

    def wait_gather(k, p):
        pltpu.make_async_copy(table.at[idx_v.at[pl.ds(C * k, C)]],
                              rows_v[p], sem_g[p]).wait()

    def write(k, p):
        pltpu.async_copy(rows_v[p], out.at[pl.ds(t0 + C * k, C)], sem_w[p])

    def wait_write(k, p):
        pltpu.make_async_copy(rows_v[p], out.at[pl.ds(t0 + C * k, C)],
                              sem_w[p]).wait()

    def add_pos(k, p):
        m = (C * k) % B
        s0 = (C * k) // B
        if m == 0:
            segs = ((0, B, s0), (B, C - B, s0 + 1))
        else:
            segs = ((0, B - m, s0), (B - m, C - (B - m), s0 + 1))

        def j_body(j, carry):
            off = LANES * j
            for start, ln, sl in segs:
                v = pos_v[sl, pl.ds(off, LANES)]
                for i in range(start, start + ln):
                    plsc.addupdate(rows_v[p].at[i, pl.ds(off, LANES)], v)
            return carry
        lax.fori_loop(0, W // LANES, j_body, 0, unroll=False)

    def chunk_tail(k, p):
        wait_gather(k, p)
        add_pos(k, p)

    gather(0, 0)
    for k in range(KFULL):
        p = k % NBUF
        if 1 <= k <= KLAST:
            write(k - 1, (k - 1) % NBUF)
        elif k >= 1:
            @pl.when(~is_last)
            def _(k=k):
                write(k - 1, (k - 1) % NBUF)
        if k + 1 < KFULL:
            def prefetch(k=k):
                if k + 1 - NBUF >= 0:
                    wait_write(k + 1 - NBUF, (k + 1) % NBUF)
                gather(k + 1, (k + 1) % NBUF)
            if k + 1 < KLAST:
                prefetch()
            else:
                pl.when(~is_last)(prefetch)
        if k < KLAST:
            chunk_tail(k, p)
        else:
            @pl.when(~is_last)
            def _():
                chunk_tail(k, p)
    @pl.when(~is_last)
    def _():
        write(KFULL - 1, (KFULL - 1) % NBUF)
        for k in range(KFULL - NBUF, KFULL):
            wait_write(k, k % NBUF)

    @pl.when(is_last)
    def _():
        for k in range(max(KLAST - NBUF, 0), KLAST):
            wait_write(k, k % NBUF)


@functools.partial(
    pl.kernel,
    out_type=jax.ShapeDtypeStruct((NT, W), jnp.float32),
    mesh=plsc.VectorSubcoreMesh(core_axis_name="c", subcore_axis_name="s"),
    scratch_types=[
        pltpu.VMEM((TPT,), jnp.int32),
        pltpu.VMEM((SPT, W), jnp.float32),
        [pltpu.VMEM((C, W), jnp.float32)] * NBUF,
        [pltpu.SemaphoreType.DMA] * NBUF,
        [pltpu.SemaphoreType.DMA] * NBUF,
    ],
)
def _sc_gather_add(table, idxs, pos_emb, out, idx_v, pos_v, rows_v, sem_g, sem_w):
    _sc_body(table, idxs, pos_emb, out, idx_v, pos_v, rows_v, sem_g, sem_w)


def kernel(Stoks, xenc, main_w, special_w, e2h_w, e2h_b, pos_emb):
    proj = _project_table(main_w, e2h_w, e2h_b)
    table = jnp.concatenate([proj, special_w], axis=0)
    pos_pad = jnp.pad(pos_emb, ((0, NWORKERS * SPT - S), (0, 0)))
    idxs = jnp.transpose(Stoks).reshape(NT).astype(jnp.int32)
    out = _sc_gather_add(table, idxs, pos_pad)
    xin = jnp.transpose(out.reshape(S, B, W), (1, 0, 2))
    return (xin.astype(xenc.dtype), 0)


# speedup vs baseline: 3.9610x; 1.1160x over previous
import functools

import jax
import jax.numpy as jnp
from jax import lax
from jax.experimental import pallas as pl
from jax.experimental.pallas import tpu as pltpu
from jax.experimental.pallas import tpu_sc as plsc

B, S = 16, 1500
CODES, SW, W = 1024, 768, 1024
NT = B * S
NWORKERS = 32
LANES = 16
TPT = 768
SPT = TPT // B
C = 24
NBUF = 3
KFULL = TPT // C
KLAST = (NT - 31 * TPT) // C


def _mm_body(a_ref, b_ref, bias_ref, sp_ref, o_ref):
    o_ref[pl.ds(0, CODES), :] = (
        jnp.dot(a_ref[...], b_ref[...], preferred_element_type=jnp.float32)
        + bias_ref[...]
    )
    o_ref[pl.ds(CODES, 8), :] = jnp.broadcast_to(sp_ref[...], (8, W))


def _project_table(main_w, e2h_w, e2h_b, special_w):
    return pl.pallas_call(
        _mm_body,
        out_shape=jax.ShapeDtypeStruct((CODES + 8, W), jnp.float32),
    )(main_w, e2h_w, e2h_b.reshape(1, W), special_w)


def _sc_body(table, idxs, pos_emb, out, idx_v, pos_v, rows_v, sem_g, sem_w):
    wid = lax.axis_index("s") * 2 + lax.axis_index("c")
    t0 = wid * TPT
    is_last = wid == NWORKERS - 1

    pltpu.sync_copy(idxs.at[pl.ds(t0, TPT)], idx_v)
    pltpu.sync_copy(pos_emb.at[pl.ds(wid * SPT, SPT)], pos_v)

    def gather(k, p):
        pltpu.async_copy(table.at[idx_v.at[pl.ds(C * k, C)]], rows_v[p], sem_g[p])

    def wait_gather(k, p):
        pltpu.make_async_copy(table.at[idx_v.at[pl.ds(C * k, C)]],
                              rows_v[p], sem_g[p]).wait()

    def write(k, p):
        pltpu.async_copy(rows_v[p], out.at[pl.ds(t0 + C * k, C)], sem_w[p])

    def wait_write(k, p):
        pltpu.make_async_copy(rows_v[p], out.at[pl.ds(t0 + C * k, C)],
                              sem_w[p]).wait()

    def add_pos(k, p):
        m = (C * k) % B
        s0 = (C * k) // B
        if m == 0:
            segs = ((0, B, s0), (B, C - B, s0 + 1))
        else:
            segs = ((0, B - m, s0), (B - m, C - (B - m), s0 + 1))

        def j_body(j, carry):
            off = LANES * j
            for start, ln, sl in segs:
                v = pos_v[sl, pl.ds(off, LANES)]
                for i in range(start, start + ln):
                    plsc.addupdate(rows_v[p].at[i, pl.ds(off, LANES)], v)
            return carry
        lax.fori_loop(0, W // LANES, j_body, 0, unroll=False)

    def chunk_tail(k, p):
        wait_gather(k, p)
        add_pos(k, p)

    gather(0, 0)
    for k in range(KFULL):
        p = k % NBUF
        if 1 <= k <= KLAST:
            write(k - 1, (k - 1) % NBUF)
        elif k >= 1:
            @pl.when(~is_last)
            def _(k=k):
                write(k - 1, (k - 1) % NBUF)
        if k + 1 < KFULL:
            def prefetch(k=k):
                if k + 1 - NBUF >= 0:
                    wait_write(k + 1 - NBUF, (k + 1) % NBUF)
                gather(k + 1, (k + 1) % NBUF)
            if k + 1 < KLAST:
                prefetch()
            else:
                pl.when(~is_last)(prefetch)
        if k < KLAST:
            chunk_tail(k, p)
        else:
            @pl.when(~is_last)
            def _():
                chunk_tail(k, p)
    @pl.when(~is_last)
    def _():
        write(KFULL - 1, (KFULL - 1) % NBUF)
        for k in range(KFULL - NBUF, KFULL):
            wait_write(k, k % NBUF)

    @pl.when(is_last)
    def _():
        for k in range(max(KLAST - NBUF, 0), KLAST):
            wait_write(k, k % NBUF)


@functools.partial(
    pl.kernel,
    out_type=jax.ShapeDtypeStruct((NT, W), jnp.float32),
    mesh=plsc.VectorSubcoreMesh(core_axis_name="c", subcore_axis_name="s"),
    scratch_types=[
        pltpu.VMEM((TPT,), jnp.int32),
        pltpu.VMEM((SPT, W), jnp.float32),
        [pltpu.VMEM((C, W), jnp.float32)] * NBUF,
        [pltpu.SemaphoreType.DMA] * NBUF,
        [pltpu.SemaphoreType.DMA] * NBUF,
    ],
)
def _sc_gather_add(table, idxs, pos_emb, out, idx_v, pos_v, rows_v, sem_g, sem_w):
    _sc_body(table, idxs, pos_emb, out, idx_v, pos_v, rows_v, sem_g, sem_w)


def kernel(Stoks, xenc, main_w, special_w, e2h_w, e2h_b, pos_emb):
    table = _project_table(main_w, e2h_w, e2h_b, special_w)
    pos_pad = jnp.pad(pos_emb, ((0, NWORKERS * SPT - S), (0, 0)))
    idxs = jnp.transpose(Stoks).reshape(NT).astype(jnp.int32)
    out = _sc_gather_add(table, idxs, pos_pad)
    xin = jnp.transpose(out.reshape(S, B, W), (1, 0, 2))
    return (xin.astype(xenc.dtype), 0)
